# Initial kernel scaffold; baseline (speedup 1.0000x reference)
import jax
import jax.numpy as jnp
from jax.experimental import pallas as pl


def kernel(node_feature, hop_features_0, nn_idx_f2v_0, nn_idx_v2f_0, etype_f2v_0, etype_v2f_0, W_nm, b_nm, W_fm, b_fm, W_v2v, b_v2v, W_f2f, b_f2f, We_f2v, W1_f2v, b1_f2v, W2_f2v, b2_f2v, We_v2f, W1_v2f, b1_v2f, W2_v2f, b2_v2f, Wc1, bc1, Wc2, bc2):
    raise NotImplementedError("write your pallas kernel here")


# speedup vs baseline: 2.9072x; 2.9072x over previous
import functools

import jax
import jax.numpy as jnp
from jax import lax
from jax.experimental import pallas as pl
from jax.experimental.pallas import tpu as pltpu
from jax.experimental.pallas import tpu_sc as plsc

_N = 50000
_F = 50000
_K = 16
_NE = 4
_D = 64

_NW = 32
_DPW = 1568
_FPAD = _NW * _DPW
_BD = 8
_BR = _BD * _K
_NBLK = _DPW // _BD

_R = 1000
_G = _F // _R


def _pre_body(node_ref, hop_ref, wnm_ref, bnm_ref, wfm_ref, bfm_ref,
              wes_ref, wf2f_ref, bf2f_ref, table_ref, base_ref):
    nnode = jnp.maximum(
        jnp.dot(node_ref[...], wnm_ref[...],
                preferred_element_type=jnp.float32) + bnm_ref[...], 0.0)
    for e in range(_NE):
        table_ref[e] = jnp.dot(nnode, wes_ref[e],
                               preferred_element_type=jnp.float32)
    nhop = jnp.maximum(
        jnp.dot(hop_ref[...], wfm_ref[...],
                preferred_element_type=jnp.float32) + bfm_ref[...], 0.0)
    base_ref[...] = jnp.dot(nhop, wf2f_ref[...],
                            preferred_element_type=jnp.float32) + bf2f_ref[...]


_pre = pl.pallas_call(
    _pre_body,
    grid=(_G,),
    in_specs=[
        pl.BlockSpec((_R, 128), lambda i: (i, 0)),
        pl.BlockSpec((_R, 128), lambda i: (i, 0)),
        pl.BlockSpec((128, _D), lambda i: (0, 0)),
        pl.BlockSpec((1, _D), lambda i: (0, 0)),
        pl.BlockSpec((128, _D), lambda i: (0, 0)),
        pl.BlockSpec((1, _D), lambda i: (0, 0)),
        pl.BlockSpec((_NE, _D, _D), lambda i: (0, 0, 0)),
        pl.BlockSpec((_D, _D), lambda i: (0, 0)),
        pl.BlockSpec((1, _D), lambda i: (0, 0)),
    ],
    out_specs=[
        pl.BlockSpec((_NE, _R, _D), lambda i: (0, i, 0)),
        pl.BlockSpec((_R, _D), lambda i: (i, 0)),
    ],
    out_shape=[
        jax.ShapeDtypeStruct((_NE, _N, _D), jnp.float32),
        jax.ShapeDtypeStruct((_F, _D), jnp.float32),
    ],
)


def _sc_body(nn_hbm, et_hbm, table_hbm, out_hbm,
             idx_v, et_v, rows_a, rows_b, out_v, sem_a, sem_b):
    cid = lax.axis_index("c")
    sid = lax.axis_index("s")
    wid = sid * 2 + cid
    ibase = wid * (_DPW * _K)
    dbase = wid * _DPW

    pltpu.sync_copy(nn_hbm.at[pl.ds(ibase, _DPW * _K)], idx_v)
    pltpu.sync_copy(et_hbm.at[pl.ds(ibase, _DPW * _K)], et_v)

    def _prep(b):
        for k in range(_BD):
            sl = pl.ds(b * _BR + k * 16, 16)
            idx_v[sl] = idx_v[sl] + et_v[sl] * _N

    def _gstart(b, rows, sem):
        pltpu.async_copy(table_hbm.at[idx_v.at[pl.ds(b * _BR, _BR)]],
                         rows, sem)

    def _gwait(b, rows, sem):
        pltpu.make_async_copy(table_hbm.at[idx_v.at[pl.ds(b * _BR, _BR)]],
                              rows, sem).wait()

    def _blk(b, rows, sem):
        _gwait(b, rows, sem)
        for d in range(_BD):
            for c in range(_D // 16):
                sl = pl.ds(c * 16, 16)
                vals = [rows[d * _K + j, sl] for j in range(_K)]
                while len(vals) > 1:
                    nxt = [vals[i] + vals[i + 1]
                           for i in range(0, len(vals) - 1, 2)]
                    if len(vals) % 2:
                        nxt.append(vals[-1])
                    vals = nxt
                out_v[d, sl] = vals[0]
        pltpu.sync_copy(out_v, out_hbm.at[pl.ds(dbase + b * _BD, _BD)])

    _prep(0)
    _gstart(0, rows_a, sem_a)
    _prep(1)
    _gstart(1, rows_b, sem_b)

    def _body(g, carry):
        b0 = g * 2

        @pl.when(b0 + 2 < _NBLK)
        def _():
            _prep(b0 + 2)

        _blk(b0, rows_a, sem_a)

        @pl.when(b0 + 2 < _NBLK)
        def _():
            _gstart(b0 + 2, rows_a, sem_a)

        @pl.when(b0 + 3 < _NBLK)
        def _():
            _prep(b0 + 3)

        _blk(b0 + 1, rows_b, sem_b)

        @pl.when(b0 + 3 < _NBLK)
        def _():
            _gstart(b0 + 3, rows_b, sem_b)

        return carry

    lax.fori_loop(0, _NBLK // 2, _body, 0)


@functools.cache
def _sc_gather_sum():
    return pl.kernel(
        _sc_body,
        out_type=jax.ShapeDtypeStruct((_FPAD, _D), jnp.float32),
        mesh=plsc.VectorSubcoreMesh(core_axis_name="c",
                                    subcore_axis_name="s"),
        compiler_params=pltpu.CompilerParams(use_tc_tiling_on_sc=False),
        scratch_types=[
            pltpu.VMEM((_DPW * _K,), jnp.int32),
            pltpu.VMEM((_DPW * _K,), jnp.int32),
            pltpu.VMEM((_BR, _D), jnp.float32),
            pltpu.VMEM((_BR, _D), jnp.float32),
            pltpu.VMEM((_BD, _D), jnp.float32),
            pltpu.SemaphoreType.DMA,
            pltpu.SemaphoreType.DMA,
        ],
    )


def _post1_body(base_ref, agg_ref, w1_ref, b1_ref, w2_ref, b2_ref,
                wc1_ref, bc1_ref, nff_ref, stats_ref):
    i = pl.program_id(0)
    agg = agg_ref[...]
    t = jnp.maximum(
        jnp.dot(agg, w1_ref[...], preferred_element_type=jnp.float32)
        + b1_ref[...], 0.0)
    nff = (base_ref[...] + agg
           + jnp.dot(t, w2_ref[...], preferred_element_type=jnp.float32)
           + b2_ref[...])
    nff_ref[...] = nff
    h = jnp.dot(nff, wc1_ref[...],
                preferred_element_type=jnp.float32) + bc1_ref[...]

    @pl.when(i == 0)
    def _():
        stats_ref[...] = jnp.zeros_like(stats_ref)

    stats_ref[0:1, :] += jnp.sum(h, axis=0, keepdims=True)
    stats_ref[1:2, :] += jnp.sum(h * h, axis=0, keepdims=True)


_post1 = pl.pallas_call(
    _post1_body,
    grid=(_G,),
    in_specs=[
        pl.BlockSpec((_R, _D), lambda i: (i, 0)),
        pl.BlockSpec((_R, _D), lambda i: (i, 0)),
        pl.BlockSpec((_D, _D), lambda i: (0, 0)),
        pl.BlockSpec((1, _D), lambda i: (0, 0)),
        pl.BlockSpec((_D, _D), lambda i: (0, 0)),
        pl.BlockSpec((1, _D), lambda i: (0, 0)),
        pl.BlockSpec((_D, 128), lambda i: (0, 0)),
        pl.BlockSpec((1, 128), lambda i: (0, 0)),
    ],
    out_specs=[
        pl.BlockSpec((_R, _D), lambda i: (i, 0)),
        pl.BlockSpec((8, 128), lambda i: (0, 0)),
    ],
    out_shape=[
        jax.ShapeDtypeStruct((_F, _D), jnp.float32),
        jax.ShapeDtypeStruct((8, 128), jnp.float32),
    ],
)


def _post2_body(nff_ref, stats_ref, wc1_ref, bc1_ref, wc2_ref, bc2_ref,
                out_ref):
    stats = stats_ref[...]
    mu = stats[0:1, :] * (1.0 / _F)
    msq = stats[1:2, :] * (1.0 / _F)
    inv = lax.rsqrt(msq - mu * mu + 1e-5)
    h = jnp.dot(nff_ref[...], wc1_ref[...],
                preferred_element_type=jnp.float32) + bc1_ref[...]
    hn = jnp.maximum((h - mu) * inv, 0.0)
    out_ref[...] = jnp.dot(hn, wc2_ref[...],
                           preferred_element_type=jnp.float32) + bc2_ref[...]


_post2 = pl.pallas_call(
    _post2_body,
    grid=(_G,),
    in_specs=[
        pl.BlockSpec((_R, _D), lambda i: (i, 0)),
        pl.BlockSpec((8, 128), lambda i: (0, 0)),
        pl.BlockSpec((_D, 128), lambda i: (0, 0)),
        pl.BlockSpec((1, 128), lambda i: (0, 0)),
        pl.BlockSpec((128, 8), lambda i: (0, 0)),
        pl.BlockSpec((1, 8), lambda i: (0, 0)),
    ],
    out_specs=pl.BlockSpec((_R, 8), lambda i: (i, 0)),
    out_shape=jax.ShapeDtypeStruct((_F, 8), jnp.float32),
)


def kernel(node_feature, hop_features_0, nn_idx_f2v_0, nn_idx_v2f_0,
           etype_f2v_0, etype_v2f_0,
           W_nm, b_nm, W_fm, b_fm, W_v2v, b_v2v, W_f2f, b_f2f,
           We_f2v, W1_f2v, b1_f2v, W2_f2v, b2_f2v,
           We_v2f, W1_v2f, b1_v2f, W2_v2f, b2_v2f,
           Wc1, bc1, Wc2, bc2):
    f32 = jnp.float32
    wes = (We_v2f * (1.0 / _K)).astype(f32)
    table, base = _pre(
        node_feature.astype(f32), hop_features_0.astype(f32),
        W_nm.astype(f32), b_nm.reshape(1, _D).astype(f32),
        W_fm.astype(f32), b_fm.reshape(1, _D).astype(f32),
        wes, W_f2f.astype(f32), b_f2f.reshape(1, _D).astype(f32))

    nn = nn_idx_v2f_0.astype(jnp.int32)
    et = etype_v2f_0.astype(jnp.int32)
    nn_p = jnp.pad(nn, ((0, _FPAD - _F), (0, 0))).reshape(-1)
    et_p = jnp.pad(et, ((0, _FPAD - _F), (0, 0))).reshape(-1)
    agg = _sc_gather_sum()(nn_p, et_p, table.reshape(_NE * _N, _D))[:_F]

    nff, stats = _post1(
        base, agg,
        W1_v2f.astype(f32), b1_v2f.reshape(1, _D).astype(f32),
        W2_v2f.astype(f32), b2_v2f.reshape(1, _D).astype(f32),
        Wc1.astype(f32), bc1.reshape(1, 128).astype(f32))

    out = _post2(
        nff, stats, Wc1.astype(f32), bc1.reshape(1, 128).astype(f32),
        jnp.pad(Wc2.astype(f32), ((0, 0), (0, 7))),
        jnp.pad(bc2.astype(f32), (0, 7)).reshape(1, 8))
    return out[:, :1]


# speedup vs baseline: 3.0829x; 1.0604x over previous
import functools

import jax
import jax.numpy as jnp
from jax import lax
from jax.experimental import pallas as pl
from jax.experimental.pallas import tpu as pltpu
from jax.experimental.pallas import tpu_sc as plsc

_N = 50000
_F = 50000
_K = 16
_NE = 4
_D = 64

_NW = 32
_DPW = 1568
_FPAD = _NW * _DPW
_BD = 8
_BR = _BD * _K
_NBLK = _DPW // _BD

_R = 1000
_G = _F // _R


def _pre_body(node_ref, hop_ref, wnm_ref, bnm_ref, wfm_ref, bfm_ref,
              wes_ref, wf2f_ref, bf2f_ref, table_ref, base_ref):
    nnode = jnp.maximum(
        jnp.dot(node_ref[...], wnm_ref[...],
                preferred_element_type=jnp.float32) + bnm_ref[...], 0.0)
    for e in range(_NE):
        table_ref[e] = jnp.dot(nnode, wes_ref[e],
                               preferred_element_type=jnp.float32)
    nhop = jnp.maximum(
        jnp.dot(hop_ref[...], wfm_ref[...],
                preferred_element_type=jnp.float32) + bfm_ref[...], 0.0)
    base_ref[...] = jnp.dot(nhop, wf2f_ref[...],
                            preferred_element_type=jnp.float32) + bf2f_ref[...]


_pre = pl.pallas_call(
    _pre_body,
    grid=(_G,),
    in_specs=[
        pl.BlockSpec((_R, 128), lambda i: (i, 0)),
        pl.BlockSpec((_R, 128), lambda i: (i, 0)),
        pl.BlockSpec((128, _D), lambda i: (0, 0)),
        pl.BlockSpec((1, _D), lambda i: (0, 0)),
        pl.BlockSpec((128, _D), lambda i: (0, 0)),
        pl.BlockSpec((1, _D), lambda i: (0, 0)),
        pl.BlockSpec((_NE, _D, _D), lambda i: (0, 0, 0)),
        pl.BlockSpec((_D, _D), lambda i: (0, 0)),
        pl.BlockSpec((1, _D), lambda i: (0, 0)),
    ],
    out_specs=[
        pl.BlockSpec((_NE, _R, _D), lambda i: (0, i, 0)),
        pl.BlockSpec((_R, _D), lambda i: (i, 0)),
    ],
    out_shape=[
        jax.ShapeDtypeStruct((_NE, _N, _D), jnp.float32),
        jax.ShapeDtypeStruct((_F, _D), jnp.float32),
    ],
)


def _sc_body(nn_hbm, et_hbm, table_hbm, out_hbm,
             idx_v, et_v, rows_a, rows_b, out_v, sem_a, sem_b):
    cid = lax.axis_index("c")
    sid = lax.axis_index("s")
    wid = sid * 2 + cid
    ibase = wid * (_DPW * _K)
    dbase = wid * _DPW

    pltpu.sync_copy(nn_hbm.at[pl.ds(ibase, _DPW * _K)], idx_v)
    pltpu.sync_copy(et_hbm.at[pl.ds(ibase, _DPW * _K)], et_v)

    def _prep(b):
        for k in range(_BD):
            sl = pl.ds(b * _BR + k * 16, 16)
            idx_v[sl] = idx_v[sl] + et_v[sl] * _N

    def _gstart(b, rows, sem):
        pltpu.async_copy(table_hbm.at[idx_v.at[pl.ds(b * _BR, _BR)]],
                         rows, sem)

    def _gwait(b, rows, sem):
        pltpu.make_async_copy(table_hbm.at[idx_v.at[pl.ds(b * _BR, _BR)]],
                              rows, sem).wait()

    def _blk(b, rows, sem):
        _gwait(b, rows, sem)
        for d in range(_BD):
            for c in range(_D // 16):
                sl = pl.ds(c * 16, 16)
                vals = [rows[d * _K + j, sl] for j in range(_K)]
                while len(vals) > 1:
                    nxt = [vals[i] + vals[i + 1]
                           for i in range(0, len(vals) - 1, 2)]
                    if len(vals) % 2:
                        nxt.append(vals[-1])
                    vals = nxt
                out_v[d, sl] = vals[0]
        pltpu.sync_copy(out_v, out_hbm.at[pl.ds(dbase + b * _BD, _BD)])

    _prep(0)
    _gstart(0, rows_a, sem_a)
    _prep(1)
    _gstart(1, rows_b, sem_b)

    def _body(g, carry):
        b0 = g * 2

        @pl.when(b0 + 2 < _NBLK)
        def _():
            _prep(b0 + 2)

        _blk(b0, rows_a, sem_a)

        @pl.when(b0 + 2 < _NBLK)
        def _():
            _gstart(b0 + 2, rows_a, sem_a)

        @pl.when(b0 + 3 < _NBLK)
        def _():
            _prep(b0 + 3)

        _blk(b0 + 1, rows_b, sem_b)

        @pl.when(b0 + 3 < _NBLK)
        def _():
            _gstart(b0 + 3, rows_b, sem_b)

        return carry

    lax.fori_loop(0, _NBLK // 2, _body, 0)


@functools.cache
def _sc_gather_sum():
    return pl.kernel(
        _sc_body,
        out_type=jax.ShapeDtypeStruct((_FPAD, _D), jnp.float32),
        mesh=plsc.VectorSubcoreMesh(core_axis_name="c",
                                    subcore_axis_name="s"),
        compiler_params=pltpu.CompilerParams(use_tc_tiling_on_sc=False),
        scratch_types=[
            pltpu.VMEM((_DPW * _K,), jnp.int32),
            pltpu.VMEM((_DPW * _K,), jnp.int32),
            pltpu.VMEM((_BR, _D), jnp.float32),
            pltpu.VMEM((_BR, _D), jnp.float32),
            pltpu.VMEM((_BD, _D), jnp.float32),
            pltpu.SemaphoreType.DMA,
            pltpu.SemaphoreType.DMA,
        ],
    )


def _post1_body(base_ref, agg_ref, w1_ref, b1_ref, w2_ref, b2_ref,
                wc1_ref, bc1_ref, nff_ref, stats_ref):
    i = pl.program_id(0)
    agg = agg_ref[...]
    t = jnp.maximum(
        jnp.dot(agg, w1_ref[...], preferred_element_type=jnp.float32)
        + b1_ref[...], 0.0)
    nff = (base_ref[...] + agg
           + jnp.dot(t, w2_ref[...], preferred_element_type=jnp.float32)
           + b2_ref[...])
    nff_ref[...] = nff
    h = jnp.dot(nff, wc1_ref[...],
                preferred_element_type=jnp.float32) + bc1_ref[...]

    @pl.when(i == 0)
    def _():
        stats_ref[...] = jnp.zeros_like(stats_ref)

    stats_ref[0:1, :] += jnp.sum(h, axis=0, keepdims=True)
    stats_ref[1:2, :] += jnp.sum(h * h, axis=0, keepdims=True)


_post1 = pl.pallas_call(
    _post1_body,
    grid=(_G,),
    in_specs=[
        pl.BlockSpec((_R, _D), lambda i: (i, 0)),
        pl.BlockSpec((_R, _D), lambda i: (i, 0)),
        pl.BlockSpec((_D, _D), lambda i: (0, 0)),
        pl.BlockSpec((1, _D), lambda i: (0, 0)),
        pl.BlockSpec((_D, _D), lambda i: (0, 0)),
        pl.BlockSpec((1, _D), lambda i: (0, 0)),
        pl.BlockSpec((_D, 128), lambda i: (0, 0)),
        pl.BlockSpec((1, 128), lambda i: (0, 0)),
    ],
    out_specs=[
        pl.BlockSpec((_R, _D), lambda i: (i, 0)),
        pl.BlockSpec((8, 128), lambda i: (0, 0)),
    ],
    out_shape=[
        jax.ShapeDtypeStruct((_F, _D), jnp.float32),
        jax.ShapeDtypeStruct((8, 128), jnp.float32),
    ],
)


def _post2_body(nff_ref, stats_ref, wc1_ref, bc1_ref, wc2_ref, bc2_ref,
                out_ref):
    stats = stats_ref[...]
    mu = stats[0:1, :] * (1.0 / _F)
    msq = stats[1:2, :] * (1.0 / _F)
    inv = lax.rsqrt(msq - mu * mu + 1e-5)
    h = jnp.dot(nff_ref[...], wc1_ref[...],
                preferred_element_type=jnp.float32) + bc1_ref[...]
    hn = jnp.maximum((h - mu) * inv, 0.0)
    out_ref[...] = jnp.dot(hn, wc2_ref[...],
                           preferred_element_type=jnp.float32) + bc2_ref[...]


_post2 = pl.pallas_call(
    _post2_body,
    grid=(_G,),
    in_specs=[
        pl.BlockSpec((_R, _D), lambda i: (i, 0)),
        pl.BlockSpec((8, 128), lambda i: (0, 0)),
        pl.BlockSpec((_D, 128), lambda i: (0, 0)),
        pl.BlockSpec((1, 128), lambda i: (0, 0)),
        pl.BlockSpec((128, 1), lambda i: (0, 0)),
        pl.BlockSpec((1, 1), lambda i: (0, 0)),
    ],
    out_specs=pl.BlockSpec((_R, 1), lambda i: (i, 0)),
    out_shape=jax.ShapeDtypeStruct((_F, 1), jnp.float32),
)


def kernel(node_feature, hop_features_0, nn_idx_f2v_0, nn_idx_v2f_0,
           etype_f2v_0, etype_v2f_0,
           W_nm, b_nm, W_fm, b_fm, W_v2v, b_v2v, W_f2f, b_f2f,
           We_f2v, W1_f2v, b1_f2v, W2_f2v, b2_f2v,
           We_v2f, W1_v2f, b1_v2f, W2_v2f, b2_v2f,
           Wc1, bc1, Wc2, bc2):
    f32 = jnp.float32
    wes = (We_v2f * (1.0 / _K)).astype(f32)
    table, base = _pre(
        node_feature.astype(f32), hop_features_0.astype(f32),
        W_nm.astype(f32), b_nm.reshape(1, _D).astype(f32),
        W_fm.astype(f32), b_fm.reshape(1, _D).astype(f32),
        wes, W_f2f.astype(f32), b_f2f.reshape(1, _D).astype(f32))

    pad1 = jnp.zeros(((_FPAD - _F) * _K,), jnp.int32)
    nn_p = jnp.concatenate([nn_idx_v2f_0.astype(jnp.int32).reshape(-1), pad1])
    et_p = jnp.concatenate([etype_v2f_0.astype(jnp.int32).reshape(-1), pad1])
    agg = _sc_gather_sum()(nn_p, et_p, table.reshape(_NE * _N, _D))

    nff, stats = _post1(
        base, agg,
        W1_v2f.astype(f32), b1_v2f.reshape(1, _D).astype(f32),
        W2_v2f.astype(f32), b2_v2f.reshape(1, _D).astype(f32),
        Wc1.astype(f32), bc1.reshape(1, 128).astype(f32))

    return _post2(
        nff, stats, Wc1.astype(f32), bc1.reshape(1, 128).astype(f32),
        Wc2.astype(f32), bc2.reshape(1, 1).astype(f32))


# speedup vs baseline: 4.3865x; 1.4229x over previous
import functools

import jax
import jax.numpy as jnp
import numpy as np
from jax import lax
from jax.experimental import pallas as pl
from jax.experimental.pallas import tpu as pltpu
from jax.experimental.pallas import tpu_sc as plsc

_PERM = np.empty((64,), np.int64)
for _c in range(2):
    for _k in range(16):
        _PERM[32 * _c + _k] = 32 * _c + 2 * _k
        _PERM[32 * _c + 16 + _k] = 32 * _c + 2 * _k + 1

_N = 50000
_F = 50000
_K = 16
_NE = 4
_D = 64

_NW = 32
_DPW = 1568
_FPAD = _NW * _DPW
_BD = 8
_BR = _BD * _K
_NBLK = _DPW // _BD

_R = 2000
_G = _F // _R


def _pre_body(node_ref, hop_ref, wnm_ref, bnm_ref, wfm_ref, bfm_ref,
              wes_ref, wf2f_ref, bf2f_ref, table_ref, base_ref):
    nnode = jnp.maximum(
        jnp.dot(node_ref[...], wnm_ref[...],
                preferred_element_type=jnp.float32) + bnm_ref[...], 0.0)
    for e in range(_NE):
        table_ref[e] = jnp.dot(
            nnode, wes_ref[e],
            preferred_element_type=jnp.float32).astype(jnp.bfloat16)
    nhop = jnp.maximum(
        jnp.dot(hop_ref[...], wfm_ref[...],
                preferred_element_type=jnp.float32) + bfm_ref[...], 0.0)
    base_ref[...] = jnp.dot(nhop, wf2f_ref[...],
                            preferred_element_type=jnp.float32) + bf2f_ref[...]


_pre = pl.pallas_call(
    _pre_body,
    grid=(_G,),
    in_specs=[
        pl.BlockSpec((_R, 128), lambda i: (i, 0)),
        pl.BlockSpec((_R, 128), lambda i: (i, 0)),
        pl.BlockSpec((128, _D), lambda i: (0, 0)),
        pl.BlockSpec((1, _D), lambda i: (0, 0)),
        pl.BlockSpec((128, _D), lambda i: (0, 0)),
        pl.BlockSpec((1, _D), lambda i: (0, 0)),
        pl.BlockSpec((_NE, _D, _D), lambda i: (0, 0, 0)),
        pl.BlockSpec((_D, _D), lambda i: (0, 0)),
        pl.BlockSpec((1, _D), lambda i: (0, 0)),
    ],
    out_specs=[
        pl.BlockSpec((_NE, _R, _D), lambda i: (0, i, 0)),
        pl.BlockSpec((_R, _D), lambda i: (i, 0)),
    ],
    out_shape=[
        jax.ShapeDtypeStruct((_NE, _N, _D), jnp.bfloat16),
        jax.ShapeDtypeStruct((_F, _D), jnp.float32),
    ],
)


def _sc_body(packed_hbm, table_hbm, out_hbm,
             idx_v, et_v, rows_a, rows_b, out_v, sem_a, sem_b):
    cid = lax.axis_index("c")
    sid = lax.axis_index("s")
    wid = sid * 2 + cid
    ibase = wid * (_DPW * _K)
    dbase = wid * _DPW

    pltpu.sync_copy(packed_hbm.at[pl.ds(ibase, _DPW * _K)], idx_v)
    pltpu.sync_copy(packed_hbm.at[pl.ds(_FPAD * _K + ibase, _DPW * _K)],
                    et_v)

    def _prep(b):
        for k in range(_BD):
            sl = pl.ds(b * _BR + k * 16, 16)
            idx_v[sl] = idx_v[sl] + et_v[sl] * _N

    def _gstart(b, rows, sem):
        pltpu.async_copy(table_hbm.at[idx_v.at[pl.ds(b * _BR, _BR)]],
                         rows, sem)

    def _gwait(b, rows, sem):
        pltpu.make_async_copy(table_hbm.at[idx_v.at[pl.ds(b * _BR, _BR)]],
                              rows, sem).wait()

    def _blk(b, rows, sem):
        _gwait(b, rows, sem)
        for d in range(_BD):
            for c in range(2):
                sl = pl.ds(c * 32, 32)
                vals = [rows[d * _K + j, sl] for j in range(_K)]
                v8 = [vals[i] + vals[i + 1] for i in range(0, 16, 2)]
                v4 = [v8[i] + v8[i + 1] for i in range(0, 8, 2)]
                ev, od = [], []
                for v in v4:
                    a, b_ = plsc.unpack(v, format=plsc.PackFormat.INTERLEAVED)
                    ev.append(a)
                    od.append(b_)
                out_v[d, pl.ds(c * 32, 16)] = (ev[0] + ev[1]) + (ev[2] + ev[3])
                out_v[d, pl.ds(c * 32 + 16, 16)] = \
                    (od[0] + od[1]) + (od[2] + od[3])
        pltpu.sync_copy(out_v, out_hbm.at[pl.ds(dbase + b * _BD, _BD)])

    _prep(0)
    _gstart(0, rows_a, sem_a)
    _prep(1)
    _gstart(1, rows_b, sem_b)

    def _body(g, carry):
        b0 = g * 2

        @pl.when(b0 + 2 < _NBLK)
        def _():
            _prep(b0 + 2)

        _blk(b0, rows_a, sem_a)

        @pl.when(b0 + 2 < _NBLK)
        def _():
            _gstart(b0 + 2, rows_a, sem_a)

        @pl.when(b0 + 3 < _NBLK)
        def _():
            _prep(b0 + 3)

        _blk(b0 + 1, rows_b, sem_b)

        @pl.when(b0 + 3 < _NBLK)
        def _():
            _gstart(b0 + 3, rows_b, sem_b)

        return carry

    lax.fori_loop(0, _NBLK // 2, _body, 0)


@functools.cache
def _sc_gather_sum():
    return pl.kernel(
        _sc_body,
        out_type=jax.ShapeDtypeStruct((_FPAD, _D), jnp.float32),
        mesh=plsc.VectorSubcoreMesh(core_axis_name="c",
                                    subcore_axis_name="s"),
        compiler_params=pltpu.CompilerParams(use_tc_tiling_on_sc=False,
                                             needs_layout_passes=False),
        scratch_types=[
            pltpu.VMEM((_DPW * _K,), jnp.int32),
            pltpu.VMEM((_DPW * _K,), jnp.int32),
            pltpu.VMEM((_BR, _D), jnp.bfloat16),
            pltpu.VMEM((_BR, _D), jnp.bfloat16),
            pltpu.VMEM((_BD, _D), jnp.float32),
            pltpu.SemaphoreType.DMA,
            pltpu.SemaphoreType.DMA,
        ],
    )


def _post1_body(base_ref, agg_ref, w1_ref, b1_ref, w2_ref, b2_ref,
                wc1_ref, bc1_ref, nff_ref, stats_ref):
    i = pl.program_id(0)
    agg = agg_ref[...]
    t = jnp.maximum(
        jnp.dot(agg, w1_ref[...], preferred_element_type=jnp.float32)
        + b1_ref[...], 0.0)
    nff = (base_ref[...] + agg
           + jnp.dot(t, w2_ref[...], preferred_element_type=jnp.float32)
           + b2_ref[...])
    nff_ref[...] = nff
    h = jnp.dot(nff, wc1_ref[...],
                preferred_element_type=jnp.float32) + bc1_ref[...]

    @pl.when(i == 0)
    def _():
        stats_ref[...] = jnp.zeros_like(stats_ref)

    stats_ref[0:1, :] += jnp.sum(h, axis=0, keepdims=True)
    stats_ref[1:2, :] += jnp.sum(h * h, axis=0, keepdims=True)


_post1 = pl.pallas_call(
    _post1_body,
    grid=(_G,),
    in_specs=[
        pl.BlockSpec((_R, _D), lambda i: (i, 0)),
        pl.BlockSpec((_R, _D), lambda i: (i, 0)),
        pl.BlockSpec((_D, _D), lambda i: (0, 0)),
        pl.BlockSpec((1, _D), lambda i: (0, 0)),
        pl.BlockSpec((_D, _D), lambda i: (0, 0)),
        pl.BlockSpec((1, _D), lambda i: (0, 0)),
        pl.BlockSpec((_D, 128), lambda i: (0, 0)),
        pl.BlockSpec((1, 128), lambda i: (0, 0)),
    ],
    out_specs=[
        pl.BlockSpec((_R, _D), lambda i: (i, 0)),
        pl.BlockSpec((8, 128), lambda i: (0, 0)),
    ],
    out_shape=[
        jax.ShapeDtypeStruct((_F, _D), jnp.float32),
        jax.ShapeDtypeStruct((8, 128), jnp.float32),
    ],
)


def _post2_body(nff_ref, stats_ref, wc1_ref, bc1_ref, wc2_ref, bc2_ref,
                out_ref):
    stats = stats_ref[...]
    mu = stats[0:1, :] * (1.0 / _F)
    msq = stats[1:2, :] * (1.0 / _F)
    inv = lax.rsqrt(msq - mu * mu + 1e-5)
    h = jnp.dot(nff_ref[...], wc1_ref[...],
                preferred_element_type=jnp.float32) + bc1_ref[...]
    hn = jnp.maximum((h - mu) * inv, 0.0)
    out_ref[...] = jnp.dot(hn, wc2_ref[...],
                           preferred_element_type=jnp.float32) + bc2_ref[...]


_post2 = pl.pallas_call(
    _post2_body,
    grid=(_G,),
    in_specs=[
        pl.BlockSpec((_R, _D), lambda i: (i, 0)),
        pl.BlockSpec((8, 128), lambda i: (0, 0)),
        pl.BlockSpec((_D, 128), lambda i: (0, 0)),
        pl.BlockSpec((1, 128), lambda i: (0, 0)),
        pl.BlockSpec((128, 1), lambda i: (0, 0)),
        pl.BlockSpec((1, 1), lambda i: (0, 0)),
    ],
    out_specs=pl.BlockSpec((_R, 1), lambda i: (i, 0)),
    out_shape=jax.ShapeDtypeStruct((_F, 1), jnp.float32),
)


def kernel(node_feature, hop_features_0, nn_idx_f2v_0, nn_idx_v2f_0,
           etype_f2v_0, etype_v2f_0,
           W_nm, b_nm, W_fm, b_fm, W_v2v, b_v2v, W_f2f, b_f2f,
           We_f2v, W1_f2v, b1_f2v, W2_f2v, b2_f2v,
           We_v2f, W1_v2f, b1_v2f, W2_v2f, b2_v2f,
           Wc1, bc1, Wc2, bc2):
    f32 = jnp.float32
    wes = (We_v2f * (1.0 / _K)).astype(f32)
    wf2f_p = W_f2f[:, _PERM].astype(f32)
    bf2f_p = b_f2f[_PERM].reshape(1, _D).astype(f32)
    w1_p = W1_v2f[_PERM, :].astype(f32)
    w2_p = W2_v2f[:, _PERM].astype(f32)
    b2_p = b2_v2f[_PERM].reshape(1, _D).astype(f32)
    wc1_p = Wc1[_PERM, :].astype(f32)

    table, base_p = _pre(
        node_feature.astype(f32), hop_features_0.astype(f32),
        W_nm.astype(f32), b_nm.reshape(1, _D).astype(f32),
        W_fm.astype(f32), b_fm.reshape(1, _D).astype(f32),
        wes, wf2f_p, bf2f_p)

    pad1 = jnp.zeros(((_FPAD - _F) * _K,), jnp.int32)
    packed = jnp.concatenate([
        nn_idx_v2f_0.astype(jnp.int32).reshape(-1), pad1,
        etype_v2f_0.astype(jnp.int32).reshape(-1), pad1])
    agg_p = _sc_gather_sum()(packed, table.reshape(_NE * _N, _D))

    nff_p, stats = _post1(
        base_p, agg_p,
        w1_p, b1_v2f.reshape(1, _D).astype(f32),
        w2_p, b2_p,
        wc1_p, bc1.reshape(1, 128).astype(f32))

    return _post2(
        nff_p, stats, wc1_p, bc1.reshape(1, 128).astype(f32),
        Wc2.astype(f32), bc2.reshape(1, 1).astype(f32))


# speedup vs baseline: 5.0594x; 1.1534x over previous
import functools

import jax
import jax.numpy as jnp
import numpy as np
from jax import lax
from jax.experimental import pallas as pl
from jax.experimental.pallas import tpu as pltpu
from jax.experimental.pallas import tpu_sc as plsc

_PERM = np.empty((64,), np.int64)
for _c in range(2):
    for _k in range(16):
        _PERM[32 * _c + _k] = 32 * _c + 2 * _k
        _PERM[32 * _c + 16 + _k] = 32 * _c + 2 * _k + 1

_N = 50000
_F = 50000
_K = 16
_NE = 4
_D = 64

_NW = 32
_DPW = 1568
_FPAD = _NW * _DPW
_BD = 8
_BR = _BD * _K
_NBLK = _DPW // _BD

_R = 2000
_G = _F // _R


def _pre_body(node_ref, hop_ref, wnm_ref, bnm_ref, wfm_ref, bfm_ref,
              wes_ref, wf2f_ref, bf2f_ref, table_ref, base_ref):
    nnode = jnp.maximum(
        jnp.dot(node_ref[...], wnm_ref[...],
                preferred_element_type=jnp.float32) + bnm_ref[...], 0.0)
    banks = [jnp.dot(nnode, wes_ref[e],
                     preferred_element_type=jnp.float32).astype(jnp.bfloat16)
             for e in range(_NE)]
    table_ref[...] = jnp.concatenate(banks, axis=1).reshape(_R // 8, 16, 128)
    nhop = jnp.maximum(
        jnp.dot(hop_ref[...], wfm_ref[...],
                preferred_element_type=jnp.float32) + bfm_ref[...], 0.0)
    base_ref[...] = jnp.dot(nhop, wf2f_ref[...],
                            preferred_element_type=jnp.float32) + bf2f_ref[...]


_pre = pl.pallas_call(
    _pre_body,
    grid=(_G,),
    in_specs=[
        pl.BlockSpec((_R, 128), lambda i: (i, 0)),
        pl.BlockSpec((_R, 128), lambda i: (i, 0)),
        pl.BlockSpec((128, _D), lambda i: (0, 0)),
        pl.BlockSpec((1, _D), lambda i: (0, 0)),
        pl.BlockSpec((128, _D), lambda i: (0, 0)),
        pl.BlockSpec((1, _D), lambda i: (0, 0)),
        pl.BlockSpec((_NE, _D, _D), lambda i: (0, 0, 0)),
        pl.BlockSpec((_D, _D), lambda i: (0, 0)),
        pl.BlockSpec((1, _D), lambda i: (0, 0)),
    ],
    out_specs=[
        pl.BlockSpec((_R // 8, 16, 128), lambda i: (i, 0, 0)),
        pl.BlockSpec((_R, _D), lambda i: (i, 0)),
    ],
    out_shape=[
        jax.ShapeDtypeStruct((_N * _NE * _D // 2048, 16, 128), jnp.bfloat16),
        jax.ShapeDtypeStruct((_F, _D), jnp.float32),
    ],
)


def _sc_body(cidx_hbm, table_hbm, out_hbm,
             idx_v, rows_a, rows_b, out_v, sem_a, sem_b):
    cid = lax.axis_index("c")
    sid = lax.axis_index("s")
    wid = sid * 2 + cid
    ibase = wid * (_DPW * _K)
    dbase = wid * _DPW

    pltpu.sync_copy(cidx_hbm.at[pl.ds(ibase, _DPW * _K)], idx_v)

    def _gstart(b, rows, sem):
        pltpu.async_copy(table_hbm.at[idx_v.at[pl.ds(b * _BR, _BR)]],
                         rows, sem)

    def _gwait(b, rows, sem):
        pltpu.make_async_copy(table_hbm.at[idx_v.at[pl.ds(b * _BR, _BR)]],
                              rows, sem).wait()

    def _blk(b, rows, sem):
        _gwait(b, rows, sem)
        for d in range(_BD):
            for c in range(2):
                sl = pl.ds(c * 32, 32)
                vals = [rows[d * _K + j, sl] for j in range(_K)]
                v8 = [vals[i] + vals[i + 1] for i in range(0, 16, 2)]
                v4 = [v8[i] + v8[i + 1] for i in range(0, 8, 2)]
                ev, od = [], []
                for v in v4:
                    a, b_ = plsc.unpack(v, format=plsc.PackFormat.INTERLEAVED)
                    ev.append(a)
                    od.append(b_)
                out_v[d, pl.ds(c * 32, 16)] = (ev[0] + ev[1]) + (ev[2] + ev[3])
                out_v[d, pl.ds(c * 32 + 16, 16)] = \
                    (od[0] + od[1]) + (od[2] + od[3])
        pltpu.sync_copy(out_v, out_hbm.at[pl.ds(dbase + b * _BD, _BD)])

    _gstart(0, rows_a, sem_a)
    _gstart(1, rows_b, sem_b)

    def _body(g, carry):
        b0 = g * 2
        _blk(b0, rows_a, sem_a)

        @pl.when(b0 + 2 < _NBLK)
        def _():
            _gstart(b0 + 2, rows_a, sem_a)

        _blk(b0 + 1, rows_b, sem_b)

        @pl.when(b0 + 3 < _NBLK)
        def _():
            _gstart(b0 + 3, rows_b, sem_b)

        return carry

    lax.fori_loop(0, _NBLK // 2, _body, 0)


@functools.cache
def _sc_gather_sum():
    return pl.kernel(
        _sc_body,
        out_type=jax.ShapeDtypeStruct((_FPAD, _D), jnp.float32),
        mesh=plsc.VectorSubcoreMesh(core_axis_name="c",
                                    subcore_axis_name="s"),
        compiler_params=pltpu.CompilerParams(use_tc_tiling_on_sc=False,
                                             needs_layout_passes=False),
        scratch_types=[
            pltpu.VMEM((_DPW * _K,), jnp.int32),
            pltpu.VMEM((_BR, _D), jnp.bfloat16),
            pltpu.VMEM((_BR, _D), jnp.bfloat16),
            pltpu.VMEM((_BD, _D), jnp.float32),
            pltpu.SemaphoreType.DMA,
            pltpu.SemaphoreType.DMA,
        ],
    )


def _post1_body(base_ref, agg_ref, w1_ref, b1_ref, w2_ref, b2_ref,
                wc1_ref, bc1_ref, nff_ref, stats_ref):
    i = pl.program_id(0)
    agg = agg_ref[...]
    t = jnp.maximum(
        jnp.dot(agg, w1_ref[...], preferred_element_type=jnp.float32)
        + b1_ref[...], 0.0)
    nff = (base_ref[...] + agg
           + jnp.dot(t, w2_ref[...], preferred_element_type=jnp.float32)
           + b2_ref[...])
    nff_ref[...] = nff
    h = jnp.dot(nff, wc1_ref[...],
                preferred_element_type=jnp.float32) + bc1_ref[...]

    @pl.when(i == 0)
    def _():
        stats_ref[...] = jnp.zeros_like(stats_ref)

    stats_ref[0:1, :] += jnp.sum(h, axis=0, keepdims=True)
    stats_ref[1:2, :] += jnp.sum(h * h, axis=0, keepdims=True)


_post1 = pl.pallas_call(
    _post1_body,
    grid=(_G,),
    in_specs=[
        pl.BlockSpec((_R, _D), lambda i: (i, 0)),
        pl.BlockSpec((_R, _D), lambda i: (i, 0)),
        pl.BlockSpec((_D, _D), lambda i: (0, 0)),
        pl.BlockSpec((1, _D), lambda i: (0, 0)),
        pl.BlockSpec((_D, _D), lambda i: (0, 0)),
        pl.BlockSpec((1, _D), lambda i: (0, 0)),
        pl.BlockSpec((_D, 128), lambda i: (0, 0)),
        pl.BlockSpec((1, 128), lambda i: (0, 0)),
    ],
    out_specs=[
        pl.BlockSpec((_R, _D), lambda i: (i, 0)),
        pl.BlockSpec((8, 128), lambda i: (0, 0)),
    ],
    out_shape=[
        jax.ShapeDtypeStruct((_F, _D), jnp.float32),
        jax.ShapeDtypeStruct((8, 128), jnp.float32),
    ],
)


def _post2_body(nff_ref, stats_ref, wc1_ref, bc1_ref, wc2_ref, bc2_ref,
                out_ref):
    stats = stats_ref[...]
    mu = stats[0:1, :] * (1.0 / _F)
    msq = stats[1:2, :] * (1.0 / _F)
    inv = lax.rsqrt(msq - mu * mu + 1e-5)
    h = jnp.dot(nff_ref[...], wc1_ref[...],
                preferred_element_type=jnp.float32) + bc1_ref[...]
    hn = jnp.maximum((h - mu) * inv, 0.0)
    out_ref[...] = jnp.dot(hn, wc2_ref[...],
                           preferred_element_type=jnp.float32) + bc2_ref[...]


_post2 = pl.pallas_call(
    _post2_body,
    grid=(_G,),
    in_specs=[
        pl.BlockSpec((_R, _D), lambda i: (i, 0)),
        pl.BlockSpec((8, 128), lambda i: (0, 0)),
        pl.BlockSpec((_D, 128), lambda i: (0, 0)),
        pl.BlockSpec((1, 128), lambda i: (0, 0)),
        pl.BlockSpec((128, 1), lambda i: (0, 0)),
        pl.BlockSpec((1, 1), lambda i: (0, 0)),
    ],
    out_specs=pl.BlockSpec((_R, 1), lambda i: (i, 0)),
    out_shape=jax.ShapeDtypeStruct((_F, 1), jnp.float32),
)


def kernel(node_feature, hop_features_0, nn_idx_f2v_0, nn_idx_v2f_0,
           etype_f2v_0, etype_v2f_0,
           W_nm, b_nm, W_fm, b_fm, W_v2v, b_v2v, W_f2f, b_f2f,
           We_f2v, W1_f2v, b1_f2v, W2_f2v, b2_f2v,
           We_v2f, W1_v2f, b1_v2f, W2_v2f, b2_v2f,
           Wc1, bc1, Wc2, bc2):
    f32 = jnp.float32
    wes = (We_v2f * (1.0 / _K)).astype(f32)
    wf2f_p = W_f2f[:, _PERM].astype(f32)
    bf2f_p = b_f2f[_PERM].reshape(1, _D).astype(f32)
    w1_p = W1_v2f[_PERM, :].astype(f32)
    w2_p = W2_v2f[:, _PERM].astype(f32)
    b2_p = b2_v2f[_PERM].reshape(1, _D).astype(f32)
    wc1_p = Wc1[_PERM, :].astype(f32)

    table, base_p = _pre(
        node_feature.astype(f32), hop_features_0.astype(f32),
        W_nm.astype(f32), b_nm.reshape(1, _D).astype(f32),
        W_fm.astype(f32), b_fm.reshape(1, _D).astype(f32),
        wes, wf2f_p, bf2f_p)

    cidx = (nn_idx_v2f_0.astype(jnp.int32) * _NE
            + etype_v2f_0.astype(jnp.int32)).reshape(-1)
    cidx = jnp.concatenate([cidx, jnp.zeros(((_FPAD - _F) * _K,), jnp.int32)])
    agg_p = _sc_gather_sum()(cidx, table.reshape(_N * _NE, _D))

    nff_p, stats = _post1(
        base_p, agg_p,
        w1_p, b1_v2f.reshape(1, _D).astype(f32),
        w2_p, b2_p,
        wc1_p, bc1.reshape(1, 128).astype(f32))

    return _post2(
        nff_p, stats, wc1_p, bc1.reshape(1, 128).astype(f32),
        Wc2.astype(f32), bc2.reshape(1, 1).astype(f32))


# speedup vs baseline: 5.6013x; 1.1071x over previous
import functools

import jax
import jax.numpy as jnp
import numpy as np
from jax import lax
from jax.experimental import pallas as pl
from jax.experimental.pallas import tpu as pltpu
from jax.experimental.pallas import tpu_sc as plsc

_PERM = np.empty((64,), np.int64)
for _c in range(2):
    for _k in range(16):
        _PERM[32 * _c + _k] = 32 * _c + 2 * _k
        _PERM[32 * _c + 16 + _k] = 32 * _c + 2 * _k + 1

_N = 50000
_F = 50000
_K = 16
_NE = 4
_D = 64

_NW = 32
_DPW = 1568
_FPAD = _NW * _DPW
_BD = 8
_BR = _BD * _K
_NBLK = _DPW // _BD

_R = 2000
_G = _F // _R


def _pre_body(node_ref, hop_ref, wnm_ref, bnm_ref, wfm_ref, bfm_ref,
              wes_ref, wf2f_ref, bf2f_ref, table_ref, base_ref):
    nnode = jnp.maximum(
        jnp.dot(node_ref[...], wnm_ref[...],
                preferred_element_type=jnp.float32) + bnm_ref[...], 0.0)
    banks = [jnp.dot(nnode, wes_ref[e],
                     preferred_element_type=jnp.float32).astype(jnp.bfloat16)
             for e in range(_NE)]
    table_ref[...] = jnp.concatenate(banks, axis=1).reshape(_R // 8, 16, 128)
    nhop = jnp.maximum(
        jnp.dot(hop_ref[...], wfm_ref[...],
                preferred_element_type=jnp.float32) + bfm_ref[...], 0.0)
    base_ref[...] = jnp.dot(nhop, wf2f_ref[...],
                            preferred_element_type=jnp.float32) + bf2f_ref[...]


_pre = pl.pallas_call(
    _pre_body,
    grid=(_G,),
    in_specs=[
        pl.BlockSpec((_R, 128), lambda i: (i, 0)),
        pl.BlockSpec((_R, 128), lambda i: (i, 0)),
        pl.BlockSpec((128, _D), lambda i: (0, 0)),
        pl.BlockSpec((1, _D), lambda i: (0, 0)),
        pl.BlockSpec((128, _D), lambda i: (0, 0)),
        pl.BlockSpec((1, _D), lambda i: (0, 0)),
        pl.BlockSpec((_NE, _D, _D), lambda i: (0, 0, 0)),
        pl.BlockSpec((_D, _D), lambda i: (0, 0)),
        pl.BlockSpec((1, _D), lambda i: (0, 0)),
    ],
    out_specs=[
        pl.BlockSpec((_R // 8, 16, 128), lambda i: (i, 0, 0)),
        pl.BlockSpec((_R, _D), lambda i: (i, 0)),
    ],
    out_shape=[
        jax.ShapeDtypeStruct((_N * _NE * _D // 2048, 16, 128), jnp.bfloat16),
        jax.ShapeDtypeStruct((_F, _D), jnp.float32),
    ],
)


def _sc_body(cidx_hbm, table_hbm, out_hbm,
             idx_v, rows0, rows1, rows2, rows3, out0, out1,
             sem0, sem1, sem2, sem3, semo0, semo1):
    cid = lax.axis_index("c")
    sid = lax.axis_index("s")
    wid = sid * 2 + cid
    ibase = wid * (_DPW * _K)
    dbase = wid * _DPW
    rows_bufs = (rows0, rows1, rows2, rows3)
    sems = (sem0, sem1, sem2, sem3)
    out_bufs = (out0, out1)
    osems = (semo0, semo1)

    pltpu.sync_copy(cidx_hbm.at[pl.ds(ibase, _DPW * _K)], idx_v)

    def _gstart(b, s):
        pltpu.async_copy(table_hbm.at[idx_v.at[pl.ds(b * _BR, _BR)]],
                         rows_bufs[s], sems[s])

    def _gwait(b, s):
        pltpu.make_async_copy(table_hbm.at[idx_v.at[pl.ds(b * _BR, _BR)]],
                              rows_bufs[s], sems[s]).wait()

    def _ostart(b, o):
        pltpu.async_copy(out_bufs[o],
                         out_hbm.at[pl.ds(dbase + b * _BD, _BD)], osems[o])

    def _owait(b, o):
        pltpu.make_async_copy(out_bufs[o],
                              out_hbm.at[pl.ds(dbase + b * _BD, _BD)],
                              osems[o]).wait()

    def _blk(b, s, o):
        _gwait(b, s)
        rows = rows_bufs[s]
        out_v = out_bufs[o]
        for d in range(_BD):
            for c in range(2):
                sl = pl.ds(c * 32, 32)
                vals = [rows[d * _K + j, sl] for j in range(_K)]
                v8 = [vals[i] + vals[i + 1] for i in range(0, 16, 2)]
                v4 = [v8[i] + v8[i + 1] for i in range(0, 8, 2)]
                ev, od = [], []
                for v in v4:
                    a, b_ = plsc.unpack(v, format=plsc.PackFormat.INTERLEAVED)
                    ev.append(a)
                    od.append(b_)
                out_v[d, pl.ds(c * 32, 16)] = (ev[0] + ev[1]) + (ev[2] + ev[3])
                out_v[d, pl.ds(c * 32 + 16, 16)] = \
                    (od[0] + od[1]) + (od[2] + od[3])
        _ostart(b, o)

    for s in range(4):
        _gstart(s, s)

    def _body(g, carry):
        b0 = g * 4
        for s in range(4):
            b = b0 + s
            o = s % 2

            @pl.when(b >= 2)
            def _():
                _owait(b - 2, o)

            _blk(b, s, o)

            @pl.when(g < (_NBLK // 4) - 1)
            def _():
                _gstart(b + 4, s)

        return carry

    lax.fori_loop(0, _NBLK // 4, _body, 0)
    _owait(_NBLK - 2, 0)
    _owait(_NBLK - 1, 1)


@functools.cache
def _sc_gather_sum():
    return pl.kernel(
        _sc_body,
        out_type=jax.ShapeDtypeStruct((_FPAD, _D), jnp.float32),
        mesh=plsc.VectorSubcoreMesh(core_axis_name="c",
                                    subcore_axis_name="s"),
        compiler_params=pltpu.CompilerParams(use_tc_tiling_on_sc=False,
                                             needs_layout_passes=False),
        scratch_types=[
            pltpu.VMEM((_DPW * _K,), jnp.int32),
            pltpu.VMEM((_BR, _D), jnp.bfloat16),
            pltpu.VMEM((_BR, _D), jnp.bfloat16),
            pltpu.VMEM((_BR, _D), jnp.bfloat16),
            pltpu.VMEM((_BR, _D), jnp.bfloat16),
            pltpu.VMEM((_BD, _D), jnp.float32),
            pltpu.VMEM((_BD, _D), jnp.float32),
            pltpu.SemaphoreType.DMA,
            pltpu.SemaphoreType.DMA,
            pltpu.SemaphoreType.DMA,
            pltpu.SemaphoreType.DMA,
            pltpu.SemaphoreType.DMA,
            pltpu.SemaphoreType.DMA,
        ],
    )


def _post1_body(base_ref, agg_ref, w1_ref, b1_ref, w2_ref, b2_ref,
                wc1_ref, bc1_ref, nff_ref, stats_ref):
    i = pl.program_id(0)
    agg = agg_ref[...]
    t = jnp.maximum(
        jnp.dot(agg, w1_ref[...], preferred_element_type=jnp.float32)
        + b1_ref[...], 0.0)
    nff = (base_ref[...] + agg
           + jnp.dot(t, w2_ref[...], preferred_element_type=jnp.float32)
           + b2_ref[...])
    nff_ref[...] = nff
    h = jnp.dot(nff, wc1_ref[...],
                preferred_element_type=jnp.float32) + bc1_ref[...]

    @pl.when(i == 0)
    def _():
        stats_ref[...] = jnp.zeros_like(stats_ref)

    stats_ref[0:1, :] += jnp.sum(h, axis=0, keepdims=True)
    stats_ref[1:2, :] += jnp.sum(h * h, axis=0, keepdims=True)


_post1 = pl.pallas_call(
    _post1_body,
    grid=(_G,),
    in_specs=[
        pl.BlockSpec((_R, _D), lambda i: (i, 0)),
        pl.BlockSpec((_R, _D), lambda i: (i, 0)),
        pl.BlockSpec((_D, _D), lambda i: (0, 0)),
        pl.BlockSpec((1, _D), lambda i: (0, 0)),
        pl.BlockSpec((_D, _D), lambda i: (0, 0)),
        pl.BlockSpec((1, _D), lambda i: (0, 0)),
        pl.BlockSpec((_D, 128), lambda i: (0, 0)),
        pl.BlockSpec((1, 128), lambda i: (0, 0)),
    ],
    out_specs=[
        pl.BlockSpec((_R, _D), lambda i: (i, 0)),
        pl.BlockSpec((8, 128), lambda i: (0, 0)),
    ],
    out_shape=[
        jax.ShapeDtypeStruct((_F, _D), jnp.float32),
        jax.ShapeDtypeStruct((8, 128), jnp.float32),
    ],
)


def _post2_body(nff_ref, stats_ref, wc1_ref, bc1_ref, wc2_ref, bc2_ref,
                out_ref):
    stats = stats_ref[...]
    mu = stats[0:1, :] * (1.0 / _F)
    msq = stats[1:2, :] * (1.0 / _F)
    inv = lax.rsqrt(msq - mu * mu + 1e-5)
    h = jnp.dot(nff_ref[...], wc1_ref[...],
                preferred_element_type=jnp.float32) + bc1_ref[...]
    hn = jnp.maximum((h - mu) * inv, 0.0)
    out_ref[...] = jnp.dot(hn, wc2_ref[...],
                           preferred_element_type=jnp.float32) + bc2_ref[...]


_post2 = pl.pallas_call(
    _post2_body,
    grid=(_G,),
    in_specs=[
        pl.BlockSpec((_R, _D), lambda i: (i, 0)),
        pl.BlockSpec((8, 128), lambda i: (0, 0)),
        pl.BlockSpec((_D, 128), lambda i: (0, 0)),
        pl.BlockSpec((1, 128), lambda i: (0, 0)),
        pl.BlockSpec((128, 1), lambda i: (0, 0)),
        pl.BlockSpec((1, 1), lambda i: (0, 0)),
    ],
    out_specs=pl.BlockSpec((_R, 1), lambda i: (i, 0)),
    out_shape=jax.ShapeDtypeStruct((_F, 1), jnp.float32),
)


def kernel(node_feature, hop_features_0, nn_idx_f2v_0, nn_idx_v2f_0,
           etype_f2v_0, etype_v2f_0,
           W_nm, b_nm, W_fm, b_fm, W_v2v, b_v2v, W_f2f, b_f2f,
           We_f2v, W1_f2v, b1_f2v, W2_f2v, b2_f2v,
           We_v2f, W1_v2f, b1_v2f, W2_v2f, b2_v2f,
           Wc1, bc1, Wc2, bc2):
    f32 = jnp.float32
    wes = (We_v2f * (1.0 / _K)).astype(f32)
    wf2f_p = W_f2f[:, _PERM].astype(f32)
    bf2f_p = b_f2f[_PERM].reshape(1, _D).astype(f32)
    w1_p = W1_v2f[_PERM, :].astype(f32)
    w2_p = W2_v2f[:, _PERM].astype(f32)
    b2_p = b2_v2f[_PERM].reshape(1, _D).astype(f32)
    wc1_p = Wc1[_PERM, :].astype(f32)

    table, base_p = _pre(
        node_feature.astype(f32), hop_features_0.astype(f32),
        W_nm.astype(f32), b_nm.reshape(1, _D).astype(f32),
        W_fm.astype(f32), b_fm.reshape(1, _D).astype(f32),
        wes, wf2f_p, bf2f_p)

    cidx = (nn_idx_v2f_0.astype(jnp.int32) * _NE
            + etype_v2f_0.astype(jnp.int32)).reshape(-1)
    cidx = jnp.concatenate([cidx, jnp.zeros(((_FPAD - _F) * _K,), jnp.int32)])
    agg_p = _sc_gather_sum()(cidx, table.reshape(_N * _NE, _D))

    nff_p, stats = _post1(
        base_p, agg_p,
        w1_p, b1_v2f.reshape(1, _D).astype(f32),
        w2_p, b2_p,
        wc1_p, bc1.reshape(1, 128).astype(f32))

    return _post2(
        nff_p, stats, wc1_p, bc1.reshape(1, 128).astype(f32),
        Wc2.astype(f32), bc2.reshape(1, 1).astype(f32))


# speedup vs baseline: 5.6775x; 1.0136x over previous
import functools

import jax
import jax.numpy as jnp
import numpy as np
from jax import lax
from jax.experimental import pallas as pl
from jax.experimental.pallas import tpu as pltpu
from jax.experimental.pallas import tpu_sc as plsc

_PERM = np.empty((64,), np.int64)
for _c in range(2):
    for _k in range(16):
        _PERM[32 * _c + _k] = 32 * _c + 2 * _k
        _PERM[32 * _c + 16 + _k] = 32 * _c + 2 * _k + 1

_N = 50000
_F = 50000
_K = 16
_NE = 4
_D = 64

_NW = 32
_DPW = 1568
_FPAD = _NW * _DPW
_BD = 8
_BR = _BD * _K
_NBLK = _DPW // _BD

_R = 2000
_G = _F // _R


def _pre_body(node_ref, hop_ref, wnm_ref, bnm_ref, wfm_ref, bfm_ref,
              wes_ref, wf2f_ref, bf2f_ref, table_ref, base_ref):
    nnode = jnp.maximum(
        jnp.dot(node_ref[...], wnm_ref[...],
                preferred_element_type=jnp.float32) + bnm_ref[...], 0.0)
    banks = [jnp.dot(nnode, wes_ref[e],
                     preferred_element_type=jnp.float32).astype(jnp.bfloat16)
             for e in range(_NE)]
    table_ref[...] = jnp.concatenate(banks, axis=1).reshape(_R // 8, 16, 128)
    nhop = jnp.maximum(
        jnp.dot(hop_ref[...], wfm_ref[...],
                preferred_element_type=jnp.float32) + bfm_ref[...], 0.0)
    base_ref[...] = jnp.dot(nhop, wf2f_ref[...],
                            preferred_element_type=jnp.float32) + bf2f_ref[...]


_pre = pl.pallas_call(
    _pre_body,
    grid=(_G,),
    in_specs=[
        pl.BlockSpec((_R, 128), lambda i: (i, 0)),
        pl.BlockSpec((_R, 128), lambda i: (i, 0)),
        pl.BlockSpec((128, _D), lambda i: (0, 0)),
        pl.BlockSpec((1, _D), lambda i: (0, 0)),
        pl.BlockSpec((128, _D), lambda i: (0, 0)),
        pl.BlockSpec((1, _D), lambda i: (0, 0)),
        pl.BlockSpec((_NE, _D, _D), lambda i: (0, 0, 0)),
        pl.BlockSpec((_D, _D), lambda i: (0, 0)),
        pl.BlockSpec((1, _D), lambda i: (0, 0)),
    ],
    out_specs=[
        pl.BlockSpec((_R // 8, 16, 128), lambda i: (i, 0, 0)),
        pl.BlockSpec((_R, _D), lambda i: (i, 0)),
    ],
    out_shape=[
        jax.ShapeDtypeStruct((_N * _NE * _D // 2048, 16, 128), jnp.bfloat16),
        jax.ShapeDtypeStruct((_F, _D), jnp.float32),
    ],
)


def _sc_body(cidx_hbm, table_hbm, out_hbm,
             idx_v, rows0, rows1, rows2, rows3, out0, out1,
             sem0, sem1, sem2, sem3, semo0, semo1):
    cid = lax.axis_index("c")
    sid = lax.axis_index("s")
    wid = sid * 2 + cid
    ibase = wid * (_DPW * _K)
    dbase = wid * _DPW
    rows_bufs = (rows0, rows1, rows2, rows3)
    sems = (sem0, sem1, sem2, sem3)
    out_bufs = (out0, out1)
    osems = (semo0, semo1)

    pltpu.sync_copy(cidx_hbm.at[pl.ds(ibase, _DPW * _K)], idx_v)

    def _gstart(b, s):
        pltpu.async_copy(table_hbm.at[idx_v.at[pl.ds(b * _BR, _BR)]],
                         rows_bufs[s], sems[s])

    def _gwait(b, s):
        pltpu.make_async_copy(table_hbm.at[idx_v.at[pl.ds(b * _BR, _BR)]],
                              rows_bufs[s], sems[s]).wait()

    def _ostart(b, o):
        pltpu.async_copy(out_bufs[o],
                         out_hbm.at[pl.ds(dbase + b * _BD, _BD)], osems[o])

    def _owait(b, o):
        pltpu.make_async_copy(out_bufs[o],
                              out_hbm.at[pl.ds(dbase + b * _BD, _BD)],
                              osems[o]).wait()

    def _blk(b, s, o):
        _gwait(b, s)
        rows = rows_bufs[s]
        out_v = out_bufs[o]
        for d in range(_BD):
            for c in range(2):
                sl = pl.ds(c * 32, 32)
                vals = [rows[d * _K + j, sl] for j in range(_K)]
                v8 = [vals[i] + vals[i + 1] for i in range(0, 16, 2)]
                v4 = [v8[i] + v8[i + 1] for i in range(0, 8, 2)]
                ev, od = [], []
                for v in v4:
                    a, b_ = plsc.unpack(v, format=plsc.PackFormat.INTERLEAVED)
                    ev.append(a)
                    od.append(b_)
                out_v[d, pl.ds(c * 32, 16)] = (ev[0] + ev[1]) + (ev[2] + ev[3])
                out_v[d, pl.ds(c * 32 + 16, 16)] = \
                    (od[0] + od[1]) + (od[2] + od[3])
        _ostart(b, o)

    for s in range(4):
        _gstart(s, s)

    def _body(g, carry):
        b0 = g * 4
        for s in range(4):
            b = b0 + s
            o = s % 2

            @pl.when(b >= 2)
            def _():
                _owait(b - 2, o)

            _blk(b, s, o)

            @pl.when(g < (_NBLK // 4) - 1)
            def _():
                _gstart(b + 4, s)

        return carry

    lax.fori_loop(0, _NBLK // 4, _body, 0)
    _owait(_NBLK - 2, 0)
    _owait(_NBLK - 1, 1)


@functools.cache
def _sc_gather_sum():
    return pl.kernel(
        _sc_body,
        out_type=jax.ShapeDtypeStruct((_FPAD, _D), jnp.float32),
        mesh=plsc.VectorSubcoreMesh(core_axis_name="c",
                                    subcore_axis_name="s"),
        compiler_params=pltpu.CompilerParams(use_tc_tiling_on_sc=False,
                                             needs_layout_passes=False),
        scratch_types=[
            pltpu.VMEM((_DPW * _K,), jnp.int32),
            pltpu.VMEM((_BR, _D), jnp.bfloat16),
            pltpu.VMEM((_BR, _D), jnp.bfloat16),
            pltpu.VMEM((_BR, _D), jnp.bfloat16),
            pltpu.VMEM((_BR, _D), jnp.bfloat16),
            pltpu.VMEM((_BD, _D), jnp.float32),
            pltpu.VMEM((_BD, _D), jnp.float32),
            pltpu.SemaphoreType.DMA,
            pltpu.SemaphoreType.DMA,
            pltpu.SemaphoreType.DMA,
            pltpu.SemaphoreType.DMA,
            pltpu.SemaphoreType.DMA,
            pltpu.SemaphoreType.DMA,
        ],
    )


def _post_body(base_ref, agg_ref, w1_ref, b1_ref, w2_ref, b2_ref,
               wc1_ref, bc1_ref, wc2_ref, bc2_ref, out_ref,
               nff_sc, stats_sc):
    p = pl.program_id(0)
    i = pl.program_id(1)

    @pl.when(p == 0)
    def _():
        agg = agg_ref[...]
        t = jnp.maximum(
            jnp.dot(agg, w1_ref[...], preferred_element_type=jnp.float32)
            + b1_ref[...], 0.0)
        nff = (base_ref[...] + agg
               + jnp.dot(t, w2_ref[...], preferred_element_type=jnp.float32)
               + b2_ref[...])
        nff_sc[pl.ds(i * _R, _R), :] = nff
        h = jnp.dot(nff, wc1_ref[...],
                    preferred_element_type=jnp.float32) + bc1_ref[...]

        @pl.when(i == 0)
        def _():
            stats_sc[...] = jnp.zeros_like(stats_sc)

        stats_sc[0:1, :] += jnp.sum(h, axis=0, keepdims=True)
        stats_sc[1:2, :] += jnp.sum(h * h, axis=0, keepdims=True)

    @pl.when(p == 1)
    def _():
        stats = stats_sc[...]
        mu = stats[0:1, :] * (1.0 / _F)
        msq = stats[1:2, :] * (1.0 / _F)
        inv = lax.rsqrt(msq - mu * mu + 1e-5)
        h = jnp.dot(nff_sc[pl.ds(i * _R, _R), :], wc1_ref[...],
                    preferred_element_type=jnp.float32) + bc1_ref[...]
        hn = jnp.maximum((h - mu) * inv, 0.0)
        out_ref[...] = jnp.dot(hn, wc2_ref[...],
                               preferred_element_type=jnp.float32) \
            + bc2_ref[...]


_post = pl.pallas_call(
    _post_body,
    grid=(2, _G),
    in_specs=[
        pl.BlockSpec((_R, _D), lambda p, i: (i * (1 - p), 0)),
        pl.BlockSpec((_R, _D), lambda p, i: (i * (1 - p), 0)),
        pl.BlockSpec((_D, _D), lambda p, i: (0, 0)),
        pl.BlockSpec((1, _D), lambda p, i: (0, 0)),
        pl.BlockSpec((_D, _D), lambda p, i: (0, 0)),
        pl.BlockSpec((1, _D), lambda p, i: (0, 0)),
        pl.BlockSpec((_D, 128), lambda p, i: (0, 0)),
        pl.BlockSpec((1, 128), lambda p, i: (0, 0)),
        pl.BlockSpec((128, 1), lambda p, i: (0, 0)),
        pl.BlockSpec((1, 1), lambda p, i: (0, 0)),
    ],
    out_specs=pl.BlockSpec((_R, 1), lambda p, i: (i, 0)),
    out_shape=jax.ShapeDtypeStruct((_F, 1), jnp.float32),
    scratch_shapes=[
        pltpu.VMEM((_F, _D), jnp.float32),
        pltpu.VMEM((8, 128), jnp.float32),
    ],
)


def kernel(node_feature, hop_features_0, nn_idx_f2v_0, nn_idx_v2f_0,
           etype_f2v_0, etype_v2f_0,
           W_nm, b_nm, W_fm, b_fm, W_v2v, b_v2v, W_f2f, b_f2f,
           We_f2v, W1_f2v, b1_f2v, W2_f2v, b2_f2v,
           We_v2f, W1_v2f, b1_v2f, W2_v2f, b2_v2f,
           Wc1, bc1, Wc2, bc2):
    f32 = jnp.float32
    wes = (We_v2f * (1.0 / _K)).astype(f32)
    wf2f_p = W_f2f[:, _PERM].astype(f32)
    bf2f_p = b_f2f[_PERM].reshape(1, _D).astype(f32)
    w1_p = W1_v2f[_PERM, :].astype(f32)
    w2_p = W2_v2f[:, _PERM].astype(f32)
    b2_p = b2_v2f[_PERM].reshape(1, _D).astype(f32)
    wc1_p = Wc1[_PERM, :].astype(f32)

    table, base_p = _pre(
        node_feature.astype(f32), hop_features_0.astype(f32),
        W_nm.astype(f32), b_nm.reshape(1, _D).astype(f32),
        W_fm.astype(f32), b_fm.reshape(1, _D).astype(f32),
        wes, wf2f_p, bf2f_p)

    cidx = (nn_idx_v2f_0.astype(jnp.int32) * _NE
            + etype_v2f_0.astype(jnp.int32)).reshape(-1)
    cidx = jnp.concatenate([cidx, jnp.zeros(((_FPAD - _F) * _K,), jnp.int32)])
    agg_p = _sc_gather_sum()(cidx, table.reshape(_N * _NE, _D))

    return _post(
        base_p, agg_p,
        w1_p, b1_v2f.reshape(1, _D).astype(f32),
        w2_p, b2_p,
        wc1_p, bc1.reshape(1, 128).astype(f32),
        Wc2.astype(f32), bc2.reshape(1, 1).astype(f32))
